# baseline (speedup 1.0000x reference)
import functools

import jax
import jax.numpy as jnp
from jax import lax
from jax.experimental import pallas as pl
from jax.experimental.pallas import tpu as pltpu
from jax.experimental.pallas import tpu_sc as plsc

N_P = 8000
N_L = 2000
N = N_P + N_L
E = 320000
D = 128
PF = 27
LF = 13
NB = 4
NG = 20
T = 1000
RMAX = 10.0

NC = 2
NS = 16
NW = NC * NS
EPW = E // NW
CH = 80
NCH = EPW // CH
ZR = 125

_F32 = jnp.float32
_I32 = jnp.int32


def _k0_body(ppos_ref, bpro_ref, pvp_ref, lpos_ref, lv_ref, blig_ref,
             tfrac_ref, wp_ref, bp_ref, wl_ref, bl_ref, w1a_ref, w1b_ref,
             b1_ref, w2_ref, wx_ref, b2_ref,
             ta_ref, tb_ref, hl_ref, pl16_ref, px_ref, py_ref, pz_ref,
             w2x_ref, c0_ref):
    ppos = ppos_ref[...]
    bpro = bpro_ref[...]
    lpos = lpos_ref[...]
    blig = blig_ref[...]

    offs = []
    for k in range(NB):
        mk = (bpro == k).astype(_F32)
        cnt = jnp.maximum(jnp.sum(mk), 1.0)
        offs.append(jnp.sum(ppos * mk, axis=0, keepdims=True) / cnt)
    off = jnp.concatenate(offs, axis=0)

    ohp = (bpro == lax.broadcasted_iota(_I32, (N_P, NB), 1)).astype(_F32)
    ohl = (blig == lax.broadcasted_iota(_I32, (N_L, NB), 1)).astype(_F32)
    pposc = ppos - jnp.dot(ohp, off, preferred_element_type=_F32)
    lposc = lpos - jnp.dot(ohl, off, preferred_element_type=_F32)

    hp = jnp.dot(pvp_ref[...], wp_ref[...], preferred_element_type=_F32) + bp_ref[...]
    tl = jnp.dot(ohl, tfrac_ref[...], preferred_element_type=_F32)
    lane16 = lax.broadcasted_iota(_I32, (N_L, 16), 1)
    ligf = (lv_ref[...] == lane16).astype(_F32) + jnp.where(lane16 == LF, tl, 0.0)
    hl = jnp.dot(ligf, wl_ref[...], preferred_element_type=_F32) + bl_ref[...]

    h = jnp.concatenate([hp, hl], axis=0)
    posc = jnp.concatenate([pposc, lposc], axis=0)

    ta_ref[...] = jnp.dot(h, w1a_ref[...], preferred_element_type=_F32)
    tb_ref[...] = jnp.dot(h, w1b_ref[...], preferred_element_type=_F32) + b1_ref[...]
    hl_ref[...] = hl
    pl16_ref[...] = jnp.concatenate([lposc, jnp.zeros((N_L, 13), _F32)], axis=1)
    post = posc.T
    px_ref[...] = post[0]
    py_ref[...] = post[1]
    pz_ref[...] = post[2]
    w2x_ref[...] = jnp.dot(w2_ref[...], wx_ref[...], preferred_element_type=_F32)
    c0_ref[...] = jnp.dot(b2_ref[...], wx_ref[...], preferred_element_type=_F32)


def _run_k0(ppos, bpro, pvp, lpos, lv, blig, tfrac, wp, bp, wl, bl,
            w1a, w1b, b1, w2, wx, b2):
    return pl.pallas_call(
        _k0_body,
        out_shape=(
            jax.ShapeDtypeStruct((N, D), _F32),
            jax.ShapeDtypeStruct((N, D), _F32),
            jax.ShapeDtypeStruct((N_L, D), _F32),
            jax.ShapeDtypeStruct((N_L, 16), _F32),
            jax.ShapeDtypeStruct((N,), _F32),
            jax.ShapeDtypeStruct((N,), _F32),
            jax.ShapeDtypeStruct((N,), _F32),
            jax.ShapeDtypeStruct((D, 1), _F32),
            jax.ShapeDtypeStruct((1, 1), _F32),
        ),
    )(ppos, bpro, pvp, lpos, lv, blig, tfrac, wp, bp, wl, bl,
      w1a, w1b, b1, w2, wx, b2)


def _k1_body(ta_hbm, tb_hbm, px_hbm, py_hbm, pz_hbm, src3_hbm, dst3_hbm,
             s_hbm, d2_hbm, rx_hbm, ry_hbm, rz_hbm,
             si_all, di_all, px_v, py_v, pz_v,
             a0, a1, o0, o1, g0, g1,
             sga0, sga1, sgb0, sgb1, sos0, sos1, sog0, sog1):
    c = lax.axis_index("c")
    s = lax.axis_index("s")
    wid = s * NC + c
    base = wid * EPW
    a_ = (a0, a1)
    o_ = (o0, o1)
    g_ = (g0, g1)
    sga_ = (sga0, sga1)
    sgb_ = (sgb0, sgb1)
    sos_ = (sos0, sos1)
    sog_ = (sog0, sog1)
    geo_hbm = (d2_hbm, rx_hbm, ry_hbm, rz_hbm)

    pltpu.sync_copy(src3_hbm.at[wid], si_all)
    pltpu.sync_copy(dst3_hbm.at[wid], di_all)
    pltpu.sync_copy(px_hbm, px_v)
    pltpu.sync_copy(py_hbm, py_v)
    pltpu.sync_copy(pz_hbm, pz_v)

    def start_gather(g, p):
        pltpu.async_copy(ta_hbm.at[si_all.at[g]], a_[p], sga_[p])
        pltpu.async_copy(tb_hbm.at[di_all.at[g]], o_[p], sgb_[p])

    def do_chunk(g, p, prefetch):
        @pl.when(g >= 1)
        def _():
            pltpu.make_async_copy(o_[1 - p], s_hbm.at[pl.ds(base, CH)],
                                  sos_[1 - p]).wait()
            for r in range(4):
                pltpu.make_async_copy(g_[1 - p].at[r],
                                      d2_hbm.at[pl.ds(base, CH)],
                                      sog_[1 - p]).wait()
        if prefetch:
            @pl.when(g + 1 < NCH)
            def _():
                start_gather(g + 1, 1 - p)
        pltpu.make_async_copy(ta_hbm.at[si_all.at[g]], a_[p], sga_[p]).wait()
        pltpu.make_async_copy(tb_hbm.at[di_all.at[g]], o_[p], sgb_[p]).wait()

        av = a_[p]
        ov = o_[p]
        gv = g_[p]

        def edge(e, _):
            for r in range(8):
                sl = pl.ds(r * 16, 16)
                plsc.addupdate(ov.at[e, sl], av[e, sl])
            return 0

        lax.fori_loop(0, CH, edge, 0, unroll=2)

        for j in range(CH // 16):
            sl = pl.ds(j * 16, 16)
            ids = si_all[g, sl]
            idd = di_all[g, sl]
            rx = plsc.load_gather(px_v, [idd]) - plsc.load_gather(px_v, [ids])
            ry = plsc.load_gather(py_v, [idd]) - plsc.load_gather(py_v, [ids])
            rz = plsc.load_gather(pz_v, [idd]) - plsc.load_gather(pz_v, [ids])
            ex = rx + 1e-12
            ey = ry + 1e-12
            ez = rz + 1e-12
            gv[0, sl] = ex * ex + ey * ey + ez * ez
            gv[1, sl] = rx
            gv[2, sl] = ry
            gv[3, sl] = rz

        pltpu.async_copy(ov, s_hbm.at[pl.ds(base + g * CH, CH)], sos_[p])
        for r in range(4):
            pltpu.async_copy(gv.at[r], geo_hbm[r].at[pl.ds(base + g * CH, CH)],
                             sog_[p])

    start_gather(0, 0)

    def pair(i, _):
        for p in range(2):
            do_chunk(i * 2 + p, p, True)
        return 0

    lax.fori_loop(0, NCH // 2, pair, 0)
    do_chunk(NCH - 1, (NCH - 1) % 2, False)
    lastp = (NCH - 1) % 2
    pltpu.make_async_copy(o_[lastp], s_hbm.at[pl.ds(base, CH)], sos_[lastp]).wait()
    for r in range(4):
        pltpu.make_async_copy(g_[lastp].at[r], d2_hbm.at[pl.ds(base, CH)],
                              sog_[lastp]).wait()


def _run_k1(ta, tb, px, py, pz, src3, dst3):
    mesh = plsc.VectorSubcoreMesh(core_axis_name="c", subcore_axis_name="s")
    f = functools.partial(
        pl.kernel,
        mesh=mesh,
        out_type=(
            jax.ShapeDtypeStruct((E, D), _F32),
            jax.ShapeDtypeStruct((E,), _F32),
            jax.ShapeDtypeStruct((E,), _F32),
            jax.ShapeDtypeStruct((E,), _F32),
            jax.ShapeDtypeStruct((E,), _F32),
        ),
        scratch_types=[
            pltpu.VMEM((NCH, CH), _I32),
            pltpu.VMEM((NCH, CH), _I32),
            pltpu.VMEM((N,), _F32),
            pltpu.VMEM((N,), _F32),
            pltpu.VMEM((N,), _F32),
            pltpu.VMEM((CH, D), _F32),
            pltpu.VMEM((CH, D), _F32),
            pltpu.VMEM((CH, D), _F32),
            pltpu.VMEM((CH, D), _F32),
            pltpu.VMEM((4, CH), _F32),
            pltpu.VMEM((4, CH), _F32),
            pltpu.SemaphoreType.DMA,
            pltpu.SemaphoreType.DMA,
            pltpu.SemaphoreType.DMA,
            pltpu.SemaphoreType.DMA,
            pltpu.SemaphoreType.DMA,
            pltpu.SemaphoreType.DMA,
            pltpu.SemaphoreType.DMA,
            pltpu.SemaphoreType.DMA,
        ],
        compiler_params=pltpu.CompilerParams(use_tc_tiling_on_sc=False,
                                             needs_layout_passes=False),
    )(_k1_body)
    return f(ta, tb, px, py, pz, src3, dst3)


BE = 12800


def _k2_body(s_ref, d2_ref, fourmu_ref, twomu2_ref, w1c_ref, w2_ref, b2_ref,
             w2x_ref, c0_ref, mo_ref, w_ref):
    i = pl.program_id(0)
    srow = s_ref[...]
    d2c = jnp.reshape(d2_ref[pl.ds(i * BE, BE)], (BE, 1))
    d2bc = jnp.dot(d2c, jnp.ones((1, 32), _F32), preferred_element_type=_F32)
    dbc = jnp.sqrt(d2bc)
    arg = dbc * fourmu_ref[...] - 2.0 * d2bc - twomu2_ref[...]
    rbf = jnp.exp(arg)
    h1 = jnp.maximum(
        srow + jnp.dot(rbf, w1c_ref[...], preferred_element_type=_F32), 0.0)
    mo_ref[...] = jnp.dot(h1, w2_ref[...], preferred_element_type=_F32) + b2_ref[...]
    coef = jnp.dot(h1, w2x_ref[...], preferred_element_type=_F32) + c0_ref[...]
    w = coef / (dbc[:, :1] + 1.0)
    w_ref[pl.ds(i * BE, BE)] = jnp.reshape(w, (BE,))


def _run_k2(s_arr, d2, fourmu, twomu2, w1c, w2, b2, w2x, c0):
    grid = (E // BE,)
    return pl.pallas_call(
        _k2_body,
        grid=grid,
        in_specs=[
            pl.BlockSpec((BE, D), lambda i: (i, 0)),
            pl.BlockSpec((E,), lambda i: (0,)),
            pl.BlockSpec((1, 32), lambda i: (0, 0)),
            pl.BlockSpec((1, 32), lambda i: (0, 0)),
            pl.BlockSpec((32, D), lambda i: (0, 0)),
            pl.BlockSpec((D, D), lambda i: (0, 0)),
            pl.BlockSpec((1, D), lambda i: (0, 0)),
            pl.BlockSpec((D, 1), lambda i: (0, 0)),
            pl.BlockSpec((1, 1), lambda i: (0, 0)),
        ],
        out_specs=(
            pl.BlockSpec((BE, D), lambda i: (i, 0)),
            pl.BlockSpec((E,), lambda i: (0,)),
        ),
        out_shape=(
            jax.ShapeDtypeStruct((E, D), _F32),
            jax.ShapeDtypeStruct((E,), _F32),
        ),
        compiler_params=pltpu.CompilerParams(
            dimension_semantics=("arbitrary",)),
    )(s_arr, d2, fourmu, twomu2, w1c, w2, b2, w2x, c0)


def _k3_body(mo_hbm, w_hbm, rx_hbm, ry_hbm, rz_hbm, dst3_hbm,
             outm_hbm, outd_hbm,
             di_all, rm0, rm1, rd0, rd1, g0, g1, accm_sh, accd_sh,
             slm0, slm1, slg0, slg1):
    c = lax.axis_index("c")
    s = lax.axis_index("s")
    wid = s * NC + c
    base = wid * EPW
    rm_ = (rm0, rm1)
    rd_ = (rd0, rd1)
    g_ = (g0, g1)
    slm_ = (slm0, slm1)
    slg_ = (slg0, slg1)
    geo_hbm = (w_hbm, rx_hbm, ry_hbm, rz_hbm)

    pltpu.sync_copy(dst3_hbm.at[wid], di_all)

    def zrow(e, _):
        for r in range(D // 16):
            rm0[e, pl.ds(r * 16, 16)] = jnp.zeros((16,), _F32)
        rd0[e, pl.ds(0, 16)] = jnp.zeros((16,), _F32)
        rd1[e, pl.ds(0, 16)] = jnp.zeros((16,), _F32)
        return 0

    lax.fori_loop(0, CH, zrow, 0)
    for k in range(7):
        pltpu.sync_copy(rm0, accm_sh.at[pl.ds(s * 625 + k * CH, CH)])
        pltpu.sync_copy(rd0, accd_sh.at[pl.ds(s * 625 + k * CH, CH)])
    pltpu.sync_copy(rm0.at[pl.ds(0, 65)], accm_sh.at[pl.ds(s * 625 + 560, 65)])
    pltpu.sync_copy(rd0.at[pl.ds(0, 65)], accd_sh.at[pl.ds(s * 625 + 560, 65)])
    plsc.subcore_barrier()

    def start_load(g, p):
        row0 = base + g * CH
        pltpu.async_copy(mo_hbm.at[pl.ds(row0, CH)], rm_[p], slm_[p])
        for r in range(4):
            pltpu.async_copy(geo_hbm[r].at[pl.ds(row0, CH)], g_[p].at[r],
                             slg_[p])

    def do_chunk(g, p, prefetch):
        if prefetch:
            @pl.when(g + 1 < NCH)
            def _():
                start_load(g + 1, 1 - p)
        pltpu.make_async_copy(mo_hbm.at[pl.ds(base, CH)], rm_[p], slm_[p]).wait()
        for r in range(4):
            pltpu.make_async_copy(w_hbm.at[pl.ds(base, CH)], g_[p].at[r],
                                  slg_[p]).wait()
        gv = g_[p]
        dv = rd_[p]
        iota16 = lax.iota(_I32, 16)
        for j in range(CH // 16):
            sl = pl.ds(j * 16, 16)
            vw = gv[0, sl]
            rows = iota16 + (j * 16)
            plsc.store_scatter(dv, [rows, jnp.zeros((16,), _I32)], gv[1, sl] * vw)
            plsc.store_scatter(dv, [rows, jnp.full((16,), 1, _I32)], gv[2, sl] * vw)
            plsc.store_scatter(dv, [rows, jnp.full((16,), 2, _I32)], gv[3, sl] * vw)
        pltpu.sync_copy(rm_[p], accm_sh.at[di_all.at[g]], add=True)
        pltpu.sync_copy(dv, accd_sh.at[di_all.at[g]], add=True)

    start_load(0, 0)

    def pair(i, _):
        for p in range(2):
            do_chunk(i * 2 + p, p, True)
        return 0

    lax.fori_loop(0, NCH // 2, pair, 0)
    do_chunk(NCH - 1, (NCH - 1) % 2, False)
    plsc.subcore_barrier()

    pltpu.sync_copy(accm_sh.at[pl.ds(N_P + s * ZR, CH)], rm0)
    pltpu.sync_copy(rm0, outm_hbm.at[c, pl.ds(s * ZR, CH)])
    pltpu.sync_copy(accm_sh.at[pl.ds(N_P + s * ZR + CH, 45)], rm1.at[pl.ds(0, 45)])
    pltpu.sync_copy(rm1.at[pl.ds(0, 45)], outm_hbm.at[c, pl.ds(s * ZR + CH, 45)])
    pltpu.sync_copy(accd_sh.at[pl.ds(N_P + s * ZR, CH)], rd0)
    pltpu.sync_copy(rd0, outd_hbm.at[c, pl.ds(s * ZR, CH)])
    pltpu.sync_copy(accd_sh.at[pl.ds(N_P + s * ZR + CH, 45)], rd1.at[pl.ds(0, 45)])
    pltpu.sync_copy(rd1.at[pl.ds(0, 45)], outd_hbm.at[c, pl.ds(s * ZR + CH, 45)])


def _run_k3(mo, w, rx, ry, rz, dst3):
    mesh = plsc.VectorSubcoreMesh(core_axis_name="c", subcore_axis_name="s")
    f = functools.partial(
        pl.kernel,
        mesh=mesh,
        out_type=(
            jax.ShapeDtypeStruct((NC, N_L, D), _F32),
            jax.ShapeDtypeStruct((NC, N_L, 16), _F32),
        ),
        scratch_types=[
            pltpu.VMEM((NCH, CH), _I32),
            pltpu.VMEM((CH, D), _F32),
            pltpu.VMEM((CH, D), _F32),
            pltpu.VMEM((CH, 16), _F32),
            pltpu.VMEM((CH, 16), _F32),
            pltpu.VMEM((4, CH), _F32),
            pltpu.VMEM((4, CH), _F32),
            pltpu.VMEM_SHARED((N, D), _F32),
            pltpu.VMEM_SHARED((N, 16), _F32),
            pltpu.SemaphoreType.DMA,
            pltpu.SemaphoreType.DMA,
            pltpu.SemaphoreType.DMA,
            pltpu.SemaphoreType.DMA,
        ],
        compiler_params=pltpu.CompilerParams(use_tc_tiling_on_sc=False,
                                             needs_layout_passes=False),
    )(_k3_body)
    return f(mo, w, rx, ry, rz, dst3)


def _k4_body(hl_ref, accm_ref, accd_ref, tal_ref, wh1_ref, wh2_ref, bh_ref,
             wv_ref, bv_ref, out_ref):
    hl = hl_ref[...]
    agg = accm_ref[0] + accm_ref[1]
    dxv = accd_ref[0] + accd_ref[1]
    pre = (jnp.dot(hl, wh1_ref[...], preferred_element_type=_F32)
           + jnp.dot(agg, wh2_ref[...], preferred_element_type=_F32)
           + bh_ref[...])
    hn = hl + jnp.maximum(pre, 0.0)
    pred = jnp.dot(hn, wv_ref[...], preferred_element_type=_F32) + bv_ref[...]
    posdx = tal_ref[...] + dxv
    lane = lax.broadcasted_iota(_I32, (N_L, 16), 1)
    out_ref[...] = jnp.where(lane < 3, posdx, pred)


def _run_k4(hl, accm, accd, tal, wh1, wh2, bh, wv, bv):
    return pl.pallas_call(
        _k4_body,
        out_shape=jax.ShapeDtypeStruct((N_L, 16), _F32),
    )(hl, accm, accd, tal, wh1, wh2, bh, wv, bv)


def kernel(protein_pos, protein_v, batch_protein, ligand_pos, ligand_v,
           batch_ligand, time_step, edge_index,
           Wp, bp, Wl, bl, W1, b1, W2, b2, Wh, bh, Wx, Wv, bv):
    pvp = jnp.pad(protein_v, ((0, 0), (0, 32 - PF)))
    wp = jnp.pad(Wp, ((0, 32 - PF), (0, 1)))
    bp_p = jnp.pad(bp, (0, 1)).reshape(1, D)
    wl = jnp.pad(Wl, ((0, 16 - (LF + 1)), (0, 1)))
    bl_p = jnp.concatenate([bl, jnp.ones((1,), _F32)]).reshape(1, D)
    w1a = W1[:D]
    w1b = W1[D:2 * D]
    w1c = jnp.pad(W1[2 * D:], ((0, 32 - NG), (0, 0)))
    b1_p = b1.reshape(1, D)
    mu = jnp.pad(jnp.linspace(0.0, RMAX, NG).astype(_F32), (0, 12)).reshape(1, 32)
    fourmu = 4.0 * mu
    twomu2 = 2.0 * mu * mu
    b2_p = b2.reshape(1, D)
    wh1 = Wh[:D]
    wh2 = Wh[D:]
    bh_p = bh.reshape(1, D)
    wv_sh = jnp.pad(Wv, ((0, 0), (3, 0)))
    bv_sh = jnp.pad(bv, (3, 0)).reshape(1, 16)
    tfrac = (time_step.astype(_F32) / T).reshape(NB, 1)
    bpro = batch_protein.astype(_I32).reshape(N_P, 1)
    blig = batch_ligand.astype(_I32).reshape(N_L, 1)
    lv = ligand_v.astype(_I32).reshape(N_L, 1)
    ei4 = edge_index.reshape(2, NW, NCH, CH)
    src3 = ei4[0]
    dst3 = ei4[1]

    ta, tb, hl, pl16, px, py, pz, w2x, c0 = _run_k0(
        protein_pos, bpro, pvp, ligand_pos, lv, blig,
        tfrac, wp, bp_p, wl, bl_p, w1a, w1b, b1_p, W2, Wx, b2_p)
    s_arr, d2, rx, ry, rz = _run_k1(ta, tb, px, py, pz, src3, dst3)
    mo, w = _run_k2(s_arr, d2, fourmu, twomu2, w1c, W2, b2_p, w2x, c0)
    accm, accd = _run_k3(mo, w, rx, ry, rz, dst3)
    return _run_k4(hl, accm, accd, pl16, wh1, wh2, bh_p, wv_sh, bv_sh)


# speedup vs baseline: 1.0597x; 1.0597x over previous
import functools

import jax
import jax.numpy as jnp
from jax import lax
from jax.experimental import pallas as pl
from jax.experimental.pallas import tpu as pltpu
from jax.experimental.pallas import tpu_sc as plsc

N_P = 8000
N_L = 2000
N = N_P + N_L
E = 320000
D = 128
PF = 27
LF = 13
NB = 4
NG = 20
T = 1000
RMAX = 10.0

NC = 2
NS = 16
NW = NC * NS
EPW = E // NW
CH = 80
NCH = EPW // CH
ZR = 125
NCA = 62
NCB = NCH - NCA
EA = NW * NCA * CH
EB = NW * NCB * CH

_F32 = jnp.float32
_I32 = jnp.int32


def _k0_body(ppos_ref, bpro_ref, pvp_ref, lpos_ref, lv_ref, blig_ref,
             tfrac_ref, wp_ref, bp_ref, wl_ref, bl_ref, w1a_ref, w1b_ref,
             b1_ref, w2_ref, wx_ref, b2_ref,
             ta_ref, tb_ref, hl_ref, pl16_ref, px_ref, py_ref, pz_ref,
             w2x_ref, c0_ref):
    ppos = ppos_ref[...]
    bpro = bpro_ref[...]
    lpos = lpos_ref[...]
    blig = blig_ref[...]

    offs = []
    for k in range(NB):
        mk = (bpro == k).astype(_F32)
        cnt = jnp.maximum(jnp.sum(mk), 1.0)
        offs.append(jnp.sum(ppos * mk, axis=0, keepdims=True) / cnt)
    off = jnp.concatenate(offs, axis=0)

    ohp = (bpro == lax.broadcasted_iota(_I32, (N_P, NB), 1)).astype(_F32)
    ohl = (blig == lax.broadcasted_iota(_I32, (N_L, NB), 1)).astype(_F32)
    pposc = ppos - jnp.dot(ohp, off, preferred_element_type=_F32)
    lposc = lpos - jnp.dot(ohl, off, preferred_element_type=_F32)

    hp = jnp.dot(pvp_ref[...], wp_ref[...], preferred_element_type=_F32) + bp_ref[...]
    tl = jnp.dot(ohl, tfrac_ref[...], preferred_element_type=_F32)
    lane16 = lax.broadcasted_iota(_I32, (N_L, 16), 1)
    ligf = (lv_ref[...] == lane16).astype(_F32) + jnp.where(lane16 == LF, tl, 0.0)
    hl = jnp.dot(ligf, wl_ref[...], preferred_element_type=_F32) + bl_ref[...]

    h = jnp.concatenate([hp, hl], axis=0)
    posc = jnp.concatenate([pposc, lposc], axis=0)

    ta_ref[...] = jnp.dot(h, w1a_ref[...], preferred_element_type=_F32)
    tb_ref[...] = jnp.dot(h, w1b_ref[...], preferred_element_type=_F32) + b1_ref[...]
    hl_ref[...] = hl
    pl16_ref[...] = jnp.concatenate([lposc, jnp.zeros((N_L, 13), _F32)], axis=1)
    post = posc.T
    px_ref[...] = post[0]
    py_ref[...] = post[1]
    pz_ref[...] = post[2]
    w2x_ref[...] = jnp.dot(w2_ref[...], wx_ref[...], preferred_element_type=_F32)
    c0_ref[...] = jnp.dot(b2_ref[...], wx_ref[...], preferred_element_type=_F32)


def _run_k0(ppos, bpro, pvp, lpos, lv, blig, tfrac, wp, bp, wl, bl,
            w1a, w1b, b1, w2, wx, b2):
    return pl.pallas_call(
        _k0_body,
        out_shape=(
            jax.ShapeDtypeStruct((N, D), _F32),
            jax.ShapeDtypeStruct((N, D), _F32),
            jax.ShapeDtypeStruct((N_L, D), _F32),
            jax.ShapeDtypeStruct((N_L, 16), _F32),
            jax.ShapeDtypeStruct((N,), _F32),
            jax.ShapeDtypeStruct((N,), _F32),
            jax.ShapeDtypeStruct((N,), _F32),
            jax.ShapeDtypeStruct((D, 1), _F32),
            jax.ShapeDtypeStruct((1, 1), _F32),
        ),
    )(ppos, bpro, pvp, lpos, lv, blig, tfrac, wp, bp, wl, bl,
      w1a, w1b, b1, w2, wx, b2)


def _make_k1_body(G0, NG):
  def _k1_body(ta_hbm, tb_hbm, px_hbm, py_hbm, pz_hbm, src3_hbm, dst3_hbm,
             s_hbm, d2_hbm, rx_hbm, ry_hbm, rz_hbm,
             si_all, di_all, px_v, py_v, pz_v,
             a0, a1, o0, o1, g0, g1,
             sga0, sga1, sgb0, sgb1, sos0, sos1, sog0, sog1):
    c = lax.axis_index("c")
    s = lax.axis_index("s")
    wid = s * NC + c
    base = wid * (NG * CH)
    a_ = (a0, a1)
    o_ = (o0, o1)
    g_ = (g0, g1)
    sga_ = (sga0, sga1)
    sgb_ = (sgb0, sgb1)
    sos_ = (sos0, sos1)
    sog_ = (sog0, sog1)
    geo_hbm = (d2_hbm, rx_hbm, ry_hbm, rz_hbm)

    pltpu.sync_copy(src3_hbm.at[wid, pl.ds(G0, NG)], si_all)
    pltpu.sync_copy(dst3_hbm.at[wid, pl.ds(G0, NG)], di_all)
    pltpu.sync_copy(px_hbm, px_v)
    pltpu.sync_copy(py_hbm, py_v)
    pltpu.sync_copy(pz_hbm, pz_v)

    def start_gather(g, p):
        pltpu.async_copy(ta_hbm.at[si_all.at[g]], a_[p], sga_[p])
        pltpu.async_copy(tb_hbm.at[di_all.at[g]], o_[p], sgb_[p])

    def do_chunk(g, p, prefetch):
        @pl.when(g >= 1)
        def _():
            pltpu.make_async_copy(o_[1 - p], s_hbm.at[pl.ds(base, CH)],
                                  sos_[1 - p]).wait()
            for r in range(4):
                pltpu.make_async_copy(g_[1 - p].at[r],
                                      d2_hbm.at[pl.ds(base, CH)],
                                      sog_[1 - p]).wait()
        if prefetch:
            @pl.when(g + 1 < NG)
            def _():
                start_gather(g + 1, 1 - p)
        pltpu.make_async_copy(ta_hbm.at[si_all.at[g]], a_[p], sga_[p]).wait()
        pltpu.make_async_copy(tb_hbm.at[di_all.at[g]], o_[p], sgb_[p]).wait()

        av = a_[p]
        ov = o_[p]
        gv = g_[p]

        def edge(e, _):
            for r in range(8):
                sl = pl.ds(r * 16, 16)
                plsc.addupdate(ov.at[e, sl], av[e, sl])
            return 0

        lax.fori_loop(0, CH, edge, 0, unroll=2)

        for j in range(CH // 16):
            sl = pl.ds(j * 16, 16)
            ids = si_all[g, sl]
            idd = di_all[g, sl]
            rx = plsc.load_gather(px_v, [idd]) - plsc.load_gather(px_v, [ids])
            ry = plsc.load_gather(py_v, [idd]) - plsc.load_gather(py_v, [ids])
            rz = plsc.load_gather(pz_v, [idd]) - plsc.load_gather(pz_v, [ids])
            ex = rx + 1e-12
            ey = ry + 1e-12
            ez = rz + 1e-12
            gv[0, sl] = ex * ex + ey * ey + ez * ez
            gv[1, sl] = rx
            gv[2, sl] = ry
            gv[3, sl] = rz

        pltpu.async_copy(ov, s_hbm.at[pl.ds(base + g * CH, CH)], sos_[p])
        for r in range(4):
            pltpu.async_copy(gv.at[r], geo_hbm[r].at[pl.ds(base + g * CH, CH)],
                             sog_[p])

    start_gather(0, 0)

    def pair(i, _):
        for p in range(2):
            do_chunk(i * 2 + p, p, True)
        return 0

    lax.fori_loop(0, NG // 2, pair, 0)
    if NG % 2:
        do_chunk(NG - 1, (NG - 1) % 2, False)
    lastp = (NG - 1) % 2
    pltpu.make_async_copy(o_[lastp], s_hbm.at[pl.ds(base, CH)], sos_[lastp]).wait()
    for r in range(4):
        pltpu.make_async_copy(g_[lastp].at[r], d2_hbm.at[pl.ds(base, CH)],
                              sog_[lastp]).wait()
  return _k1_body


def _run_k1(ta, tb, px, py, pz, src3, dst3, g0c, ngc):
    eh = NW * ngc * CH
    mesh = plsc.VectorSubcoreMesh(core_axis_name="c", subcore_axis_name="s")
    f = functools.partial(
        pl.kernel,
        mesh=mesh,
        out_type=(
            jax.ShapeDtypeStruct((eh, D), _F32),
            jax.ShapeDtypeStruct((eh,), _F32),
            jax.ShapeDtypeStruct((eh,), _F32),
            jax.ShapeDtypeStruct((eh,), _F32),
            jax.ShapeDtypeStruct((eh,), _F32),
        ),
        scratch_types=[
            pltpu.VMEM((ngc, CH), _I32),
            pltpu.VMEM((ngc, CH), _I32),
            pltpu.VMEM((N,), _F32),
            pltpu.VMEM((N,), _F32),
            pltpu.VMEM((N,), _F32),
            pltpu.VMEM((CH, D), _F32),
            pltpu.VMEM((CH, D), _F32),
            pltpu.VMEM((CH, D), _F32),
            pltpu.VMEM((CH, D), _F32),
            pltpu.VMEM((4, CH), _F32),
            pltpu.VMEM((4, CH), _F32),
            pltpu.SemaphoreType.DMA,
            pltpu.SemaphoreType.DMA,
            pltpu.SemaphoreType.DMA,
            pltpu.SemaphoreType.DMA,
            pltpu.SemaphoreType.DMA,
            pltpu.SemaphoreType.DMA,
            pltpu.SemaphoreType.DMA,
            pltpu.SemaphoreType.DMA,
        ],
        compiler_params=pltpu.CompilerParams(use_tc_tiling_on_sc=False,
                                             needs_layout_passes=False),
    )(_make_k1_body(g0c, ngc))
    return f(ta, tb, px, py, pz, src3, dst3)


BE = 2560


def _k2_body(s_ref, d2_ref, fourmu_ref, twomu2_ref, w1c_ref, w2_ref, b2_ref,
             w2x_ref, c0_ref, mo_ref, w_ref):
    i = pl.program_id(0)
    srow = s_ref[...]
    d2c = jnp.reshape(d2_ref[pl.ds(i * BE, BE)], (BE, 1))
    d2bc = jnp.dot(d2c, jnp.ones((1, 32), _F32), preferred_element_type=_F32)
    dbc = jnp.sqrt(d2bc)
    arg = dbc * fourmu_ref[...] - 2.0 * d2bc - twomu2_ref[...]
    rbf = jnp.exp(arg)
    h1 = jnp.maximum(
        srow + jnp.dot(rbf, w1c_ref[...], preferred_element_type=_F32), 0.0)
    mo_ref[...] = jnp.dot(h1, w2_ref[...], preferred_element_type=_F32) + b2_ref[...]
    coef = jnp.dot(h1, w2x_ref[...], preferred_element_type=_F32) + c0_ref[...]
    w = coef / (dbc[:, :1] + 1.0)
    w_ref[pl.ds(i * BE, BE)] = jnp.reshape(w, (BE,))


def _run_k2(s_arr, d2, fourmu, twomu2, w1c, w2, b2, w2x, c0, eh):
    grid = (eh // BE,)
    return pl.pallas_call(
        _k2_body,
        grid=grid,
        in_specs=[
            pl.BlockSpec((BE, D), lambda i: (i, 0)),
            pl.BlockSpec((eh,), lambda i: (0,)),
            pl.BlockSpec((1, 32), lambda i: (0, 0)),
            pl.BlockSpec((1, 32), lambda i: (0, 0)),
            pl.BlockSpec((32, D), lambda i: (0, 0)),
            pl.BlockSpec((D, D), lambda i: (0, 0)),
            pl.BlockSpec((1, D), lambda i: (0, 0)),
            pl.BlockSpec((D, 1), lambda i: (0, 0)),
            pl.BlockSpec((1, 1), lambda i: (0, 0)),
        ],
        out_specs=(
            pl.BlockSpec((BE, D), lambda i: (i, 0)),
            pl.BlockSpec((eh,), lambda i: (0,)),
        ),
        out_shape=(
            jax.ShapeDtypeStruct((eh, D), _F32),
            jax.ShapeDtypeStruct((eh,), _F32),
        ),
        compiler_params=pltpu.CompilerParams(
            dimension_semantics=("arbitrary",)),
    )(s_arr, d2, fourmu, twomu2, w1c, w2, b2, w2x, c0)


def _k3_body(moA_hbm, moB_hbm, wA_hbm, rxA_hbm, ryA_hbm, rzA_hbm,
             wB_hbm, rxB_hbm, ryB_hbm, rzB_hbm, dst3_hbm,
             outm_hbm, outd_hbm,
             di_all, rm0, rm1, rd0, rd1, g0, g1, accm_sh, accd_sh,
             slm0, slm1, slg0, slg1):
    c = lax.axis_index("c")
    s = lax.axis_index("s")
    wid = s * NC + c
    rm_ = (rm0, rm1)
    rd_ = (rd0, rd1)
    g_ = (g0, g1)
    slm_ = (slm0, slm1)
    slg_ = (slg0, slg1)

    pltpu.sync_copy(dst3_hbm.at[wid], di_all)

    def zrow(e, _):
        for r in range(D // 16):
            rm0[e, pl.ds(r * 16, 16)] = jnp.zeros((16,), _F32)
        rd0[e, pl.ds(0, 16)] = jnp.zeros((16,), _F32)
        rd1[e, pl.ds(0, 16)] = jnp.zeros((16,), _F32)
        return 0

    lax.fori_loop(0, CH, zrow, 0)
    for k in range(7):
        pltpu.sync_copy(rm0, accm_sh.at[pl.ds(s * 625 + k * CH, CH)])
        pltpu.sync_copy(rd0, accd_sh.at[pl.ds(s * 625 + k * CH, CH)])
    pltpu.sync_copy(rm0.at[pl.ds(0, 65)], accm_sh.at[pl.ds(s * 625 + 560, 65)])
    pltpu.sync_copy(rd0.at[pl.ds(0, 65)], accd_sh.at[pl.ds(s * 625 + 560, 65)])
    plsc.subcore_barrier()

    def run_half(mo_hbm, w_hbm, rx_hbm, ry_hbm, rz_hbm, gg0, ng):
        base = wid * (ng * CH)
        geo_hbm = (w_hbm, rx_hbm, ry_hbm, rz_hbm)

        def start_load(g, p):
            row0 = base + g * CH
            pltpu.async_copy(mo_hbm.at[pl.ds(row0, CH)], rm_[p], slm_[p])
            for r in range(4):
                pltpu.async_copy(geo_hbm[r].at[pl.ds(row0, CH)], g_[p].at[r],
                                 slg_[p])

        def do_chunk(g, p, prefetch):
            if prefetch:
                @pl.when(g + 1 < ng)
                def _():
                    start_load(g + 1, 1 - p)
            pltpu.make_async_copy(mo_hbm.at[pl.ds(base, CH)], rm_[p],
                                  slm_[p]).wait()
            for r in range(4):
                pltpu.make_async_copy(w_hbm.at[pl.ds(base, CH)], g_[p].at[r],
                                      slg_[p]).wait()
            gv = g_[p]
            dv = rd_[p]
            iota16 = lax.iota(_I32, 16)
            for j in range(CH // 16):
                sl = pl.ds(j * 16, 16)
                vw = gv[0, sl]
                rows = iota16 + (j * 16)
                plsc.store_scatter(dv, [rows, jnp.zeros((16,), _I32)],
                                   gv[1, sl] * vw)
                plsc.store_scatter(dv, [rows, jnp.full((16,), 1, _I32)],
                                   gv[2, sl] * vw)
                plsc.store_scatter(dv, [rows, jnp.full((16,), 2, _I32)],
                                   gv[3, sl] * vw)
            pltpu.sync_copy(rm_[p], accm_sh.at[di_all.at[gg0 + g]], add=True)
            pltpu.sync_copy(dv, accd_sh.at[di_all.at[gg0 + g]], add=True)

        start_load(0, 0)

        def pair(i, _):
            for p in range(2):
                do_chunk(i * 2 + p, p, True)
            return 0

        lax.fori_loop(0, ng // 2, pair, 0)
        if ng % 2:
            do_chunk(ng - 1, (ng - 1) % 2, False)

    run_half(moA_hbm, wA_hbm, rxA_hbm, ryA_hbm, rzA_hbm, 0, NCA)
    run_half(moB_hbm, wB_hbm, rxB_hbm, ryB_hbm, rzB_hbm, NCA, NCB)
    plsc.subcore_barrier()

    pltpu.sync_copy(accm_sh.at[pl.ds(N_P + s * ZR, CH)], rm0)
    pltpu.sync_copy(rm0, outm_hbm.at[c, pl.ds(s * ZR, CH)])
    pltpu.sync_copy(accm_sh.at[pl.ds(N_P + s * ZR + CH, 45)], rm1.at[pl.ds(0, 45)])
    pltpu.sync_copy(rm1.at[pl.ds(0, 45)], outm_hbm.at[c, pl.ds(s * ZR + CH, 45)])
    pltpu.sync_copy(accd_sh.at[pl.ds(N_P + s * ZR, CH)], rd0)
    pltpu.sync_copy(rd0, outd_hbm.at[c, pl.ds(s * ZR, CH)])
    pltpu.sync_copy(accd_sh.at[pl.ds(N_P + s * ZR + CH, 45)], rd1.at[pl.ds(0, 45)])
    pltpu.sync_copy(rd1.at[pl.ds(0, 45)], outd_hbm.at[c, pl.ds(s * ZR + CH, 45)])


def _run_k3(moA, moB, wA, rxA, ryA, rzA, wB, rxB, ryB, rzB, dst3):
    mesh = plsc.VectorSubcoreMesh(core_axis_name="c", subcore_axis_name="s")
    f = functools.partial(
        pl.kernel,
        mesh=mesh,
        out_type=(
            jax.ShapeDtypeStruct((NC, N_L, D), _F32),
            jax.ShapeDtypeStruct((NC, N_L, 16), _F32),
        ),
        scratch_types=[
            pltpu.VMEM((NCH, CH), _I32),
            pltpu.VMEM((CH, D), _F32),
            pltpu.VMEM((CH, D), _F32),
            pltpu.VMEM((CH, 16), _F32),
            pltpu.VMEM((CH, 16), _F32),
            pltpu.VMEM((4, CH), _F32),
            pltpu.VMEM((4, CH), _F32),
            pltpu.VMEM_SHARED((N, D), _F32),
            pltpu.VMEM_SHARED((N, 16), _F32),
            pltpu.SemaphoreType.DMA,
            pltpu.SemaphoreType.DMA,
            pltpu.SemaphoreType.DMA,
            pltpu.SemaphoreType.DMA,
        ],
        compiler_params=pltpu.CompilerParams(use_tc_tiling_on_sc=False,
                                             needs_layout_passes=False),
    )(_k3_body)
    return f(moA, moB, wA, rxA, ryA, rzA, wB, rxB, ryB, rzB, dst3)


def _k4_body(hl_ref, accm_ref, accd_ref, tal_ref, wh1_ref, wh2_ref, bh_ref,
             wv_ref, bv_ref, out_ref):
    hl = hl_ref[...]
    agg = accm_ref[0] + accm_ref[1]
    dxv = accd_ref[0] + accd_ref[1]
    pre = (jnp.dot(hl, wh1_ref[...], preferred_element_type=_F32)
           + jnp.dot(agg, wh2_ref[...], preferred_element_type=_F32)
           + bh_ref[...])
    hn = hl + jnp.maximum(pre, 0.0)
    pred = jnp.dot(hn, wv_ref[...], preferred_element_type=_F32) + bv_ref[...]
    posdx = tal_ref[...] + dxv
    lane = lax.broadcasted_iota(_I32, (N_L, 16), 1)
    out_ref[...] = jnp.where(lane < 3, posdx, pred)


def _run_k4(hl, accm, accd, tal, wh1, wh2, bh, wv, bv):
    return pl.pallas_call(
        _k4_body,
        out_shape=jax.ShapeDtypeStruct((N_L, 16), _F32),
    )(hl, accm, accd, tal, wh1, wh2, bh, wv, bv)


def kernel(protein_pos, protein_v, batch_protein, ligand_pos, ligand_v,
           batch_ligand, time_step, edge_index,
           Wp, bp, Wl, bl, W1, b1, W2, b2, Wh, bh, Wx, Wv, bv):
    pvp = jnp.pad(protein_v, ((0, 0), (0, 32 - PF)))
    wp = jnp.pad(Wp, ((0, 32 - PF), (0, 1)))
    bp_p = jnp.pad(bp, (0, 1)).reshape(1, D)
    wl = jnp.pad(Wl, ((0, 16 - (LF + 1)), (0, 1)))
    bl_p = jnp.concatenate([bl, jnp.ones((1,), _F32)]).reshape(1, D)
    w1a = W1[:D]
    w1b = W1[D:2 * D]
    w1c = jnp.pad(W1[2 * D:], ((0, 32 - NG), (0, 0)))
    b1_p = b1.reshape(1, D)
    mu = jnp.pad(jnp.linspace(0.0, RMAX, NG).astype(_F32), (0, 12)).reshape(1, 32)
    fourmu = 4.0 * mu
    twomu2 = 2.0 * mu * mu
    b2_p = b2.reshape(1, D)
    wh1 = Wh[:D]
    wh2 = Wh[D:]
    bh_p = bh.reshape(1, D)
    wv_sh = jnp.pad(Wv, ((0, 0), (3, 0)))
    bv_sh = jnp.pad(bv, (3, 0)).reshape(1, 16)
    tfrac = (time_step.astype(_F32) / T).reshape(NB, 1)
    bpro = batch_protein.astype(_I32).reshape(N_P, 1)
    blig = batch_ligand.astype(_I32).reshape(N_L, 1)
    lv = ligand_v.astype(_I32).reshape(N_L, 1)
    ei4 = edge_index.reshape(2, NW, NCH, CH)
    src3 = ei4[0]
    dst3 = ei4[1]

    ta, tb, hl, pl16, px, py, pz, w2x, c0 = _run_k0(
        protein_pos, bpro, pvp, ligand_pos, lv, blig,
        tfrac, wp, bp_p, wl, bl_p, w1a, w1b, b1_p, W2, Wx, b2_p)
    sA, d2A, rxA, ryA, rzA = _run_k1(ta, tb, px, py, pz, src3, dst3, 0, NCA)
    sB, d2B, rxB, ryB, rzB = _run_k1(ta, tb, px, py, pz, src3, dst3, NCA, NCB)
    moA, wA = _run_k2(sA, d2A, fourmu, twomu2, w1c, W2, b2_p, w2x, c0, EA)
    moB, wB = _run_k2(sB, d2B, fourmu, twomu2, w1c, W2, b2_p, w2x, c0, EB)
    accm, accd = _run_k3(moA, moB, wA, rxA, ryA, rzA, wB, rxB, ryB, rzB, dst3)
    return _run_k4(hl, accm, accd, pl16, wh1, wh2, bh_p, wv_sh, bv_sh)
